# baseline (speedup 1.0000x reference)
import jax
import jax.numpy as jnp
from jax.experimental import pallas as pl

B, W, H, A, C = 16, 64, 64, 3, 80
N = W * H * A
CHUNK = 2048
CS = CHUNK // 128
NB = N // CHUNK


def _loss_body(conf_ref, mask_ref, iou_ref, sw_ref, swrow_ref, pxy_ref,
               pwh_ref, tb_ref, cls_ref, ts_ref, out_ref):
    b = pl.program_id(0)
    j = pl.program_id(1)

    conf = conf_ref[0, 0]
    mask = mask_ref[0, 0]
    iou = iou_ref[0, 0]
    sw = sw_ref[0, 0]

    noobj_p = 0.25 * jnp.sum(jnp.where(mask == 0.0, conf * conf, 0.0))
    obj_p = 0.5 * jnp.sum(jnp.where(mask == 1.0, (conf - iou) ** 2, 0.0))

    s = jax.lax.broadcasted_iota(jnp.int32, (CS, 128), 0)
    l = jax.lax.broadcasted_iota(jnp.int32, (CS, 128), 1)
    n = j * CHUNK + s * 128 + l
    fmx = ((n & 4095) >> 6).astype(jnp.float32)
    fmy = (n & 63).astype(jnp.float32)

    gtsw = jnp.where(sw > 0.0, sw, 0.0)

    x0 = pxy_ref[0, 0, 0]
    x1 = pxy_ref[0, 1, 0]
    w0 = pwh_ref[0, 0, 0]
    w1 = pwh_ref[0, 1, 0]
    t0 = tb_ref[0, 0, 0]
    t1 = tb_ref[0, 1, 0]
    t2 = tb_ref[0, 2, 0]
    t3 = tb_ref[0, 3, 0]

    def bce(x, t):
        return jnp.maximum(x, 0.0) - x * t + jnp.log1p(jnp.exp(-jnp.abs(x)))

    xy_p = 0.5 * jnp.sum(gtsw * (bce(x0 - fmx, t0 - fmx)
                                 + bce(x1 - fmy, t1 - fmy)))
    wh_p = 0.5 * jnp.sum(gtsw * ((w0 - t2) ** 2 + (w1 - t3) ** 2))

    d = cls_ref[0, 0] - ts_ref[0, 0]
    gt_row = (swrow_ref[0, 0] > 0.0).astype(jnp.float32)
    score_p = 0.5 * jnp.sum(
        jax.lax.dot(gt_row, d * d, preferred_element_type=jnp.float32))

    row = jax.lax.broadcasted_iota(jnp.int32, (8, 128), 0)
    lane = jax.lax.broadcasted_iota(jnp.int32, (8, 128), 1)
    on0 = row == 0
    partial = (jnp.where(on0 & (lane == 0), noobj_p, 0.0)
               + jnp.where(on0 & (lane == 1), obj_p, 0.0)
               + jnp.where(on0 & (lane == 2), xy_p, 0.0)
               + jnp.where(on0 & (lane == 3), wh_p, 0.0)
               + jnp.where(on0 & (lane == 4), score_p, 0.0))

    first = (b == 0) & (j == 0)

    @pl.when(first)
    def _():
        out_ref[...] = partial

    @pl.when(jnp.logical_not(first))
    def _():
        out_ref[...] += partial


def kernel(epoch, conf, pred_xy, pred_wh, cls_score, cls_out, obj_mask,
           true_bbox, true_score, pred_gt_iou, scale_weight):
    conf_r = conf.reshape(B, NB, CS, 128)
    mask_r = obj_mask.reshape(B, NB, CS, 128)
    iou_r = pred_gt_iou.reshape(B, NB, CS, 128)
    sw_r = scale_weight.reshape(B, NB, CS, 128)
    sw_row = scale_weight.reshape(B, NB, 1, CHUNK)
    pxy_t = pred_xy.reshape(B, N, 2).transpose(0, 2, 1).reshape(
        B, 2, NB, CS, 128)
    pwh_t = pred_wh.reshape(B, N, 2).transpose(0, 2, 1).reshape(
        B, 2, NB, CS, 128)
    tb_t = true_bbox.reshape(B, N, 4).transpose(0, 2, 1).reshape(
        B, 4, NB, CS, 128)
    cls_r = cls_score.reshape(B, NB, CHUNK, C)
    ts_r = true_score.reshape(B, NB, CHUNK, C)

    small_spec = pl.BlockSpec((1, 1, CS, 128), lambda b, j: (b, j, 0, 0))
    row_spec = pl.BlockSpec((1, 1, 1, CHUNK), lambda b, j: (b, j, 0, 0))
    xy_spec = pl.BlockSpec((1, 2, 1, CS, 128), lambda b, j: (b, 0, j, 0, 0))
    tb_spec = pl.BlockSpec((1, 4, 1, CS, 128), lambda b, j: (b, 0, j, 0, 0))
    big_spec = pl.BlockSpec((1, 1, CHUNK, C), lambda b, j: (b, j, 0, 0))

    acc = pl.pallas_call(
        _loss_body,
        grid=(B, NB),
        in_specs=[small_spec, small_spec, small_spec, small_spec, row_spec,
                  xy_spec, xy_spec, tb_spec, big_spec, big_spec],
        out_specs=pl.BlockSpec((8, 128), lambda b, j: (0, 0)),
        out_shape=jax.ShapeDtypeStruct((8, 128), jnp.float32),
    )(conf_r, mask_r, iou_r, sw_r, sw_row, pxy_t, pwh_t, tb_t, cls_r, ts_r)

    noobj_loss = acc[0, 0] / B
    obj_loss = acc[0, 1] / B
    xy_loss = acc[0, 2] / B
    wh_loss = acc[0, 3] / B
    score_loss = acc[0, 4] / B
    return (score_loss, noobj_loss / 4.0, obj_loss / 4.0, score_loss / 4.0,
            xy_loss / 4.0, wh_loss / 4.0)


# speedup vs baseline: 5.1875x; 5.1875x over previous
import jax
import jax.numpy as jnp
from jax.experimental import pallas as pl

B, W, H, A, C = 16, 64, 64, 3, 80
WB = 16
WC = W // WB


def _loss_body(conf_ref, mask_ref, iou_ref, sw_ref, pxy_ref, pwh_ref,
               tb_ref, cls_ref, ts_ref, out_ref):
    b = pl.program_id(0)
    wc = pl.program_id(1)

    conf = conf_ref[0]
    mask = mask_ref[0]
    iou = iou_ref[0]
    sw = sw_ref[0]

    noobj_p = 0.25 * jnp.sum(jnp.where(mask == 0.0, conf * conf, 0.0))
    obj_p = 0.5 * jnp.sum(jnp.where(mask == 1.0, (conf - iou) ** 2, 0.0))

    shape = (WB, A, 1, H)
    wi = jax.lax.broadcasted_iota(jnp.int32, shape, 0)
    a = jax.lax.broadcasted_iota(jnp.int32, shape, 1)
    h = jax.lax.broadcasted_iota(jnp.int32, shape, 3)
    n = (wc * WB + wi) * (H * A) + h * A + a
    fmx = ((n & 4095) >> 6).astype(jnp.float32)
    fmy = (n & 63).astype(jnp.float32)

    gtsw = jnp.where(sw > 0.0, sw, 0.0)

    x0 = pxy_ref[0, :, :, 0:1, :]
    x1 = pxy_ref[0, :, :, 1:2, :]
    w0 = pwh_ref[0, :, :, 0:1, :]
    w1 = pwh_ref[0, :, :, 1:2, :]
    t0 = tb_ref[0, :, :, 0:1, :]
    t1 = tb_ref[0, :, :, 1:2, :]
    t2 = tb_ref[0, :, :, 2:3, :]
    t3 = tb_ref[0, :, :, 3:4, :]

    def bce(x, t):
        return jnp.maximum(x, 0.0) - x * t + jnp.log1p(jnp.exp(-jnp.abs(x)))

    xy_p = 0.5 * jnp.sum(gtsw * (bce(x0 - fmx, t0 - fmx)
                                 + bce(x1 - fmy, t1 - fmy)))
    wh_p = 0.5 * jnp.sum(gtsw * ((w0 - t2) ** 2 + (w1 - t3) ** 2))

    d = cls_ref[0] - ts_ref[0]
    gt = (sw > 0.0).astype(jnp.float32)
    masked = jax.lax.dot_general(
        gt.reshape(WB * A, 1, H), (d * d).reshape(WB * A, H, C),
        dimension_numbers=(((2,), (1,)), ((0,), (0,))),
        preferred_element_type=jnp.float32)
    score_p = 0.5 * jnp.sum(masked)

    row = jax.lax.broadcasted_iota(jnp.int32, (8, 128), 0)
    lane = jax.lax.broadcasted_iota(jnp.int32, (8, 128), 1)
    on0 = row == 0
    partial = (jnp.where(on0 & (lane == 0), noobj_p, 0.0)
               + jnp.where(on0 & (lane == 1), obj_p, 0.0)
               + jnp.where(on0 & (lane == 2), xy_p, 0.0)
               + jnp.where(on0 & (lane == 3), wh_p, 0.0)
               + jnp.where(on0 & (lane == 4), score_p, 0.0))

    first = (b == 0) & (wc == 0)

    @pl.when(first)
    def _():
        out_ref[...] = partial

    @pl.when(jnp.logical_not(first))
    def _():
        out_ref[...] += partial


def kernel(epoch, conf, pred_xy, pred_wh, cls_score, cls_out, obj_mask,
           true_bbox, true_score, pred_gt_iou, scale_weight):
    nat = lambda x: x.transpose(0, 1, 3, 4, 2)
    conf_n = nat(conf)
    mask_n = nat(obj_mask)
    iou_n = nat(pred_gt_iou)
    sw_n = nat(scale_weight)
    pxy_n = nat(pred_xy)
    pwh_n = nat(pred_wh)
    tb_n = nat(true_bbox)
    cls_n = cls_score.transpose(0, 1, 3, 2, 4)
    ts_n = true_score.transpose(0, 1, 3, 2, 4)

    def spec(k):
        return pl.BlockSpec((1, WB, A, k, H), lambda b, wc: (b, wc, 0, 0, 0))

    big_spec = pl.BlockSpec((1, WB, A, H, C), lambda b, wc: (b, wc, 0, 0, 0))

    acc = pl.pallas_call(
        _loss_body,
        grid=(B, WC),
        in_specs=[spec(1), spec(1), spec(1), spec(1), spec(2), spec(2),
                  spec(4), big_spec, big_spec],
        out_specs=pl.BlockSpec((8, 128), lambda b, wc: (0, 0)),
        out_shape=jax.ShapeDtypeStruct((8, 128), jnp.float32),
    )(conf_n, mask_n, iou_n, sw_n, pxy_n, pwh_n, tb_n, cls_n, ts_n)

    noobj_loss = acc[0, 0] / B
    obj_loss = acc[0, 1] / B
    xy_loss = acc[0, 2] / B
    wh_loss = acc[0, 3] / B
    score_loss = acc[0, 4] / B
    return (score_loss, noobj_loss / 4.0, obj_loss / 4.0, score_loss / 4.0,
            xy_loss / 4.0, wh_loss / 4.0)


# speedup vs baseline: 5.6039x; 1.0803x over previous
import jax
import jax.numpy as jnp
from jax import lax
from jax.experimental import pallas as pl
from jax.experimental.pallas import tpu as pltpu
from jax.experimental.pallas import tpu_sc as plsc

B, W, H, A, C = 16, 64, 64, 3, 80
WB = 64
WC = W // WB

NC, NS, L = 2, 16, 16
NW = NC * NS
WSC = W // 2


def _tc_body(sw_ref, pxy_ref, pwh_ref, tb_ref, cls_ref, ts_ref, out_ref):
    b = pl.program_id(0)
    wc = pl.program_id(1)

    sw = sw_ref[0]

    shape = (WB, A, 1, H)
    wi = jax.lax.broadcasted_iota(jnp.int32, shape, 0)
    a = jax.lax.broadcasted_iota(jnp.int32, shape, 1)
    h = jax.lax.broadcasted_iota(jnp.int32, shape, 3)
    n = (wc * WB + wi) * (H * A) + h * A + a
    fmx = ((n & 4095) >> 6).astype(jnp.float32)
    fmy = (n & 63).astype(jnp.float32)

    gtsw = jnp.where(sw > 0.0, sw, 0.0)

    x0 = pxy_ref[0, :, :, 0:1, :]
    x1 = pxy_ref[0, :, :, 1:2, :]
    w0 = pwh_ref[0, :, :, 0:1, :]
    w1 = pwh_ref[0, :, :, 1:2, :]
    t0 = tb_ref[0, :, :, 0:1, :]
    t1 = tb_ref[0, :, :, 1:2, :]
    t2 = tb_ref[0, :, :, 2:3, :]
    t3 = tb_ref[0, :, :, 3:4, :]

    def bce(x, t):
        return jnp.maximum(x, 0.0) - x * t + jnp.log1p(jnp.exp(-jnp.abs(x)))

    xy_p = 0.5 * jnp.sum(gtsw * (bce(x0 - fmx, t0 - fmx)
                                 + bce(x1 - fmy, t1 - fmy)))
    wh_p = 0.5 * jnp.sum(gtsw * ((w0 - t2) ** 2 + (w1 - t3) ** 2))

    d = cls_ref[0] - ts_ref[0]
    gt = (sw > 0.0).astype(jnp.float32)
    masked = jax.lax.dot_general(
        gt.reshape(WB * A, 1, H), (d * d).reshape(WB * A, H, C),
        dimension_numbers=(((2,), (1,)), ((0,), (0,))),
        preferred_element_type=jnp.float32)
    score_p = 0.5 * jnp.sum(masked)

    row = jax.lax.broadcasted_iota(jnp.int32, (8, 128), 0)
    lane = jax.lax.broadcasted_iota(jnp.int32, (8, 128), 1)
    on0 = row == 0
    partial = (jnp.where(on0 & (lane == 2), xy_p, 0.0)
               + jnp.where(on0 & (lane == 3), wh_p, 0.0)
               + jnp.where(on0 & (lane == 4), score_p, 0.0))

    first = (b == 0) & (wc == 0)

    @pl.when(first)
    def _():
        out_ref[...] = partial

    @pl.when(jnp.logical_not(first))
    def _():
        out_ref[...] += partial


def _sc_body(conf_hbm, mask_hbm, iou_hbm, noobj_out, obj_out,
             conf_v, mask_v, iou_v, res_v):
    wid = lax.axis_index("s") * NC + lax.axis_index("c")
    b = wid // 2
    w0 = (wid % 2) * WSC

    pltpu.sync_copy(conf_hbm.at[b, pl.ds(w0, WSC)], conf_v)
    pltpu.sync_copy(mask_hbm.at[b, pl.ds(w0, WSC)], mask_v)
    pltpu.sync_copy(iou_hbm.at[b, pl.ds(w0, WSC)], iou_v)

    nacc = jnp.zeros((L,), jnp.float32)
    oacc = jnp.zeros((L,), jnp.float32)
    for w2 in range(WSC):
        for a2 in range(A):
            for j in range(H // L):
                c = conf_v[w2, a2, 0, pl.ds(j * L, L)]
                m = mask_v[w2, a2, 0, pl.ds(j * L, L)]
                u = iou_v[w2, a2, 0, pl.ds(j * L, L)]
                du = c - u
                nacc = nacc + jnp.where(m == 0.0, c * c, 0.0)
                oacc = oacc + jnp.where(m == 1.0, du * du, 0.0)

    res_v[0, :] = 0.25 * nacc
    res_v[1, :] = 0.5 * oacc
    pltpu.sync_copy(res_v.at[0], noobj_out.at[wid])
    pltpu.sync_copy(res_v.at[1], obj_out.at[wid])


def kernel(epoch, conf, pred_xy, pred_wh, cls_score, cls_out, obj_mask,
           true_bbox, true_score, pred_gt_iou, scale_weight):
    nat = lambda x: x.transpose(0, 1, 3, 4, 2)
    conf_n = nat(conf)
    mask_n = nat(obj_mask)
    iou_n = nat(pred_gt_iou)
    sw_n = nat(scale_weight)
    pxy_n = nat(pred_xy)
    pwh_n = nat(pred_wh)
    tb_n = nat(true_bbox)
    cls_n = cls_score.transpose(0, 1, 3, 2, 4)
    ts_n = true_score.transpose(0, 1, 3, 2, 4)

    sc_loss = pl.kernel(
        _sc_body,
        out_type=(jax.ShapeDtypeStruct((NW, L), jnp.float32),
                  jax.ShapeDtypeStruct((NW, L), jnp.float32)),
        mesh=plsc.VectorSubcoreMesh(core_axis_name="c", subcore_axis_name="s"),
        scratch_types=[pltpu.VMEM((WSC, A, 1, H), jnp.float32),
                       pltpu.VMEM((WSC, A, 1, H), jnp.float32),
                       pltpu.VMEM((WSC, A, 1, H), jnp.float32),
                       pltpu.VMEM((2, L), jnp.float32)],
    )
    noobj_parts, obj_parts = sc_loss(conf_n, mask_n, iou_n)

    def spec(k):
        return pl.BlockSpec((1, WB, A, k, H), lambda b, wc: (b, wc, 0, 0, 0))

    big_spec = pl.BlockSpec((1, WB, A, H, C), lambda b, wc: (b, wc, 0, 0, 0))

    acc = pl.pallas_call(
        _tc_body,
        grid=(B, WC),
        in_specs=[spec(1), spec(2), spec(2), spec(4), big_spec, big_spec],
        out_specs=pl.BlockSpec((8, 128), lambda b, wc: (0, 0)),
        out_shape=jax.ShapeDtypeStruct((8, 128), jnp.float32),
    )(sw_n, pxy_n, pwh_n, tb_n, cls_n, ts_n)

    noobj_loss = jnp.sum(noobj_parts) / B
    obj_loss = jnp.sum(obj_parts) / B
    xy_loss = acc[0, 2] / B
    wh_loss = acc[0, 3] / B
    score_loss = acc[0, 4] / B
    return (score_loss, noobj_loss / 4.0, obj_loss / 4.0, score_loss / 4.0,
            xy_loss / 4.0, wh_loss / 4.0)


# speedup vs baseline: 5.6754x; 1.0128x over previous
import jax
import jax.numpy as jnp
from jax import lax
from jax.experimental import pallas as pl
from jax.experimental.pallas import tpu as pltpu
from jax.experimental.pallas import tpu_sc as plsc

B, W, H, A, C = 16, 64, 64, 3, 80
WB = 64
WC = W // WB

NC, NS, L = 2, 16, 16
NW = NC * NS
WSC = W // 2


def _tc_body(conf_ref, mask_ref, iou_ref, sw_ref, pxy_ref, pwh_ref,
             tb_ref, cls_ref, ts_ref, out_ref):
    del conf_ref, mask_ref, iou_ref
    b = pl.program_id(0)
    wc = pl.program_id(1)

    sw = sw_ref[0]

    shape = (WB, A, 1, H)
    wi = jax.lax.broadcasted_iota(jnp.int32, shape, 0)
    a = jax.lax.broadcasted_iota(jnp.int32, shape, 1)
    h = jax.lax.broadcasted_iota(jnp.int32, shape, 3)
    n = (wc * WB + wi) * (H * A) + h * A + a
    fmx = ((n & 4095) >> 6).astype(jnp.float32)
    fmy = (n & 63).astype(jnp.float32)

    gtsw = jnp.where(sw > 0.0, sw, 0.0)

    x0 = pxy_ref[0, :, :, 0:1, :]
    x1 = pxy_ref[0, :, :, 1:2, :]
    w0 = pwh_ref[0, :, :, 0:1, :]
    w1 = pwh_ref[0, :, :, 1:2, :]
    t0 = tb_ref[0, :, :, 0:1, :]
    t1 = tb_ref[0, :, :, 1:2, :]
    t2 = tb_ref[0, :, :, 2:3, :]
    t3 = tb_ref[0, :, :, 3:4, :]

    def bce(x, t):
        return jnp.maximum(x, 0.0) - x * t + jnp.log1p(jnp.exp(-jnp.abs(x)))

    xy_p = 0.5 * jnp.sum(gtsw * (bce(x0 - fmx, t0 - fmx)
                                 + bce(x1 - fmy, t1 - fmy)))
    wh_p = 0.5 * jnp.sum(gtsw * ((w0 - t2) ** 2 + (w1 - t3) ** 2))

    d = cls_ref[0] - ts_ref[0]
    gt = (sw > 0.0).astype(jnp.float32)
    masked = jax.lax.dot_general(
        gt.reshape(WB * A, 1, H), (d * d).reshape(WB * A, H, C),
        dimension_numbers=(((2,), (1,)), ((0,), (0,))),
        preferred_element_type=jnp.float32)
    score_p = 0.5 * jnp.sum(masked)

    row = jax.lax.broadcasted_iota(jnp.int32, (8, 128), 0)
    lane = jax.lax.broadcasted_iota(jnp.int32, (8, 128), 1)
    on0 = row == 0
    partial = (jnp.where(on0 & (lane == 2), xy_p, 0.0)
               + jnp.where(on0 & (lane == 3), wh_p, 0.0)
               + jnp.where(on0 & (lane == 4), score_p, 0.0))

    first = (b == 0) & (wc == 0)

    @pl.when(first)
    def _():
        out_ref[...] = partial

    @pl.when(jnp.logical_not(first))
    def _():
        out_ref[...] += partial


def _sc_body(conf_hbm, mask_hbm, iou_hbm, out_hbm,
             conf_v, mask_v, iou_v, res_v):
    wid = lax.axis_index("s") * NC + lax.axis_index("c")
    b = wid // 2
    w0 = (wid % 2) * WSC

    pltpu.sync_copy(conf_hbm.at[b, pl.ds(w0, WSC)], conf_v)
    pltpu.sync_copy(mask_hbm.at[b, pl.ds(w0, WSC)], mask_v)
    pltpu.sync_copy(iou_hbm.at[b, pl.ds(w0, WSC)], iou_v)

    nacc = jnp.zeros((L,), jnp.float32)
    oacc = jnp.zeros((L,), jnp.float32)
    for w2 in range(WSC):
        for a2 in range(A):
            for j in range(H // L):
                c = conf_v[w2, a2, 0, pl.ds(j * L, L)]
                m = mask_v[w2, a2, 0, pl.ds(j * L, L)]
                u = iou_v[w2, a2, 0, pl.ds(j * L, L)]
                du = c - u
                nacc = nacc + jnp.where(m == 0.0, c * c, 0.0)
                oacc = oacc + jnp.where(m == 1.0, du * du, 0.0)

    res_v[0, :] = 0.25 * nacc
    res_v[1, :] = 0.5 * oacc
    pltpu.sync_copy(res_v.at[0], out_hbm.at[0, wid])
    pltpu.sync_copy(res_v.at[1], out_hbm.at[1, wid])


def kernel(epoch, conf, pred_xy, pred_wh, cls_score, cls_out, obj_mask,
           true_bbox, true_score, pred_gt_iou, scale_weight):
    nat = lambda x: x.transpose(0, 1, 3, 4, 2)
    conf_n = nat(conf)
    mask_n = nat(obj_mask)
    iou_n = nat(pred_gt_iou)
    sw_n = nat(scale_weight)
    pxy_n = nat(pred_xy)
    pwh_n = nat(pred_wh)
    tb_n = nat(true_bbox)
    cls_n = cls_score.transpose(0, 1, 3, 2, 4)
    ts_n = true_score.transpose(0, 1, 3, 2, 4)

    sc_loss = pl.kernel(
        _sc_body,
        out_type=jax.ShapeDtypeStruct((2, NW, L), jnp.float32),
        mesh=plsc.VectorSubcoreMesh(core_axis_name="c", subcore_axis_name="s"),
        scratch_types=[pltpu.VMEM((WSC, A, 1, H), jnp.float32),
                       pltpu.VMEM((WSC, A, 1, H), jnp.float32),
                       pltpu.VMEM((WSC, A, 1, H), jnp.float32),
                       pltpu.VMEM((2, L), jnp.float32)],
    )
    conf_parts = sc_loss(conf_n, mask_n, iou_n)

    def spec(k):
        return pl.BlockSpec((1, WB, A, k, H), lambda b, wc: (b, wc, 0, 0, 0))

    tiny_spec = pl.BlockSpec((1, 1, A, 1, H), lambda b, wc: (b, 0, 0, 0, 0))
    big_spec = pl.BlockSpec((1, WB, A, H, C), lambda b, wc: (b, wc, 0, 0, 0))

    acc = pl.pallas_call(
        _tc_body,
        grid=(B, WC),
        in_specs=[tiny_spec, tiny_spec, tiny_spec, spec(1), spec(2), spec(2),
                  spec(4), big_spec, big_spec],
        out_specs=pl.BlockSpec((8, 128), lambda b, wc: (0, 0)),
        out_shape=jax.ShapeDtypeStruct((8, 128), jnp.float32),
    )(conf_n, mask_n, iou_n, sw_n, pxy_n, pwh_n, tb_n, cls_n, ts_n)

    conf_losses = jnp.sum(conf_parts, axis=(1, 2))
    noobj_loss = conf_losses[0] / B
    obj_loss = conf_losses[1] / B
    xy_loss = acc[0, 2] / B
    wh_loss = acc[0, 3] / B
    score_loss = acc[0, 4] / B
    return (score_loss, noobj_loss / 4.0, obj_loss / 4.0, score_loss / 4.0,
            xy_loss / 4.0, wh_loss / 4.0)


# speedup vs baseline: 8.0344x; 1.4156x over previous
import jax
import jax.numpy as jnp
from jax.experimental import pallas as pl
from jax.experimental.pallas import tpu as pltpu

B, W, H, A, C = 16, 64, 64, 3, 80
WB = 64
WC = W // WB


def _loss_body(conf_ref, mask_ref, iou_ref, sw_ref, pxy_ref, pwh_ref,
               tb_ref, cls_ref, ts_ref,
               total_ref, noobj_ref, obj_ref, score_ref, xy_ref, wh_ref,
               acc_ref):
    b = pl.program_id(0)
    wc = pl.program_id(1)

    conf = conf_ref[0]
    mask = mask_ref[0]
    iou = iou_ref[0]
    sw = sw_ref[0]

    noobj_p = 0.25 * jnp.sum(jnp.where(mask == 0.0, conf * conf, 0.0))
    obj_p = 0.5 * jnp.sum(jnp.where(mask == 1.0, (conf - iou) ** 2, 0.0))

    shape = (WB, A, 1, H)
    wi = jax.lax.broadcasted_iota(jnp.int32, shape, 0)
    a = jax.lax.broadcasted_iota(jnp.int32, shape, 1)
    h = jax.lax.broadcasted_iota(jnp.int32, shape, 3)
    n = (wc * WB + wi) * (H * A) + h * A + a
    fmx = ((n & 4095) >> 6).astype(jnp.float32)
    fmy = (n & 63).astype(jnp.float32)

    gtsw = jnp.where(sw > 0.0, sw, 0.0)

    x0 = pxy_ref[0, :, :, 0:1, :]
    x1 = pxy_ref[0, :, :, 1:2, :]
    w0 = pwh_ref[0, :, :, 0:1, :]
    w1 = pwh_ref[0, :, :, 1:2, :]
    t0 = tb_ref[0, :, :, 0:1, :]
    t1 = tb_ref[0, :, :, 1:2, :]
    t2 = tb_ref[0, :, :, 2:3, :]
    t3 = tb_ref[0, :, :, 3:4, :]

    def bce(x, t):
        return jnp.maximum(x, 0.0) - x * t + jnp.log1p(jnp.exp(-jnp.abs(x)))

    xy_p = 0.5 * jnp.sum(gtsw * (bce(x0 - fmx, t0 - fmx)
                                 + bce(x1 - fmy, t1 - fmy)))
    wh_p = 0.5 * jnp.sum(gtsw * ((w0 - t2) ** 2 + (w1 - t3) ** 2))

    d = cls_ref[0] - ts_ref[0]
    gt = (sw > 0.0).astype(jnp.float32)
    masked = jax.lax.dot_general(
        gt.reshape(WB * A, 1, H), (d * d).reshape(WB * A, H, C),
        dimension_numbers=(((2,), (1,)), ((0,), (0,))),
        preferred_element_type=jnp.float32)
    score_p = 0.5 * jnp.sum(masked)

    first = (b == 0) & (wc == 0)
    last = (b == B - 1) & (wc == WC - 1)

    @pl.when(first)
    def _():
        acc_ref[0] = noobj_p
        acc_ref[1] = obj_p
        acc_ref[2] = xy_p
        acc_ref[3] = wh_p
        acc_ref[4] = score_p

    @pl.when(jnp.logical_not(first))
    def _():
        acc_ref[0] += noobj_p
        acc_ref[1] += obj_p
        acc_ref[2] += xy_p
        acc_ref[3] += wh_p
        acc_ref[4] += score_p

    @pl.when(last)
    def _():
        inv_b = 1.0 / B
        score_loss = acc_ref[4] * inv_b
        total_ref[0] = score_loss
        noobj_ref[0] = acc_ref[0] * (inv_b / 4.0)
        obj_ref[0] = acc_ref[1] * (inv_b / 4.0)
        score_ref[0] = score_loss / 4.0
        xy_ref[0] = acc_ref[2] * (inv_b / 4.0)
        wh_ref[0] = acc_ref[3] * (inv_b / 4.0)


def kernel(epoch, conf, pred_xy, pred_wh, cls_score, cls_out, obj_mask,
           true_bbox, true_score, pred_gt_iou, scale_weight):
    nat = lambda x: x.transpose(0, 1, 3, 4, 2)
    conf_n = nat(conf)
    mask_n = nat(obj_mask)
    iou_n = nat(pred_gt_iou)
    sw_n = nat(scale_weight)
    pxy_n = nat(pred_xy)
    pwh_n = nat(pred_wh)
    tb_n = nat(true_bbox)
    cls_n = cls_score.transpose(0, 1, 3, 2, 4)
    ts_n = true_score.transpose(0, 1, 3, 2, 4)

    def spec(k):
        return pl.BlockSpec((1, WB, A, k, H), lambda b, wc: (b, wc, 0, 0, 0))

    big_spec = pl.BlockSpec((1, WB, A, H, C), lambda b, wc: (b, wc, 0, 0, 0))
    scalar_out = pl.BlockSpec(memory_space=pltpu.SMEM)
    out_sds = jax.ShapeDtypeStruct((1,), jnp.float32)

    outs = pl.pallas_call(
        _loss_body,
        grid=(B, WC),
        in_specs=[spec(1), spec(1), spec(1), spec(1), spec(2), spec(2),
                  spec(4), big_spec, big_spec],
        out_specs=[scalar_out] * 6,
        out_shape=[out_sds] * 6,
        scratch_shapes=[pltpu.SMEM((5,), jnp.float32)],
    )(conf_n, mask_n, iou_n, sw_n, pxy_n, pwh_n, tb_n, cls_n, ts_n)

    total, noobj_loss, obj_loss, score_loss, xy_loss, wh_loss = outs
    return (total[0], noobj_loss[0], obj_loss[0], score_loss[0],
            xy_loss[0], wh_loss[0])
